# Initial kernel scaffold; baseline (speedup 1.0000x reference)
import jax
import jax.numpy as jnp
from jax.experimental import pallas as pl


def kernel(node_features, edge_index, W0, a_src0, a_dst0, prelu0_alpha, W1, a_src1, a_dst1, W2, a_src2, a_dst2, final_prelu_alpha):
    raise NotImplementedError("write your pallas kernel here")


# speedup vs baseline: 5.5471x; 5.5471x over previous
import functools

import jax
import jax.numpy as jnp
from jax import lax
from jax.experimental import pallas as pl
from jax.experimental.pallas import tpu as pltpu
from jax.experimental.pallas import tpu_sc as plsc

_N = 10000
_D = 256
_H = 256
_NPAD = 10240
_R = 512
_NSC = 2
_NTS = 16
_K = 128
_RSLICE = _NPAD // _NTS
_CB = 32
_DR = _NPAD // 16
_DSL = _DR // _NTS


def _tc_proj(x_pad, w_all, a_all):
    def body(x_ref, w_ref, a_ref, h_ref, s_ref):
        cid = pl.program_id(2)
        xb = x_ref[...]
        hb = jnp.dot(xb, w_ref[0, 0], preferred_element_type=jnp.float32)
        h_ref[0, 0] = hb
        sb = lax.dot_general(a_ref[0, 0], hb, (((1,), (1,)), ((), ())),
                             preferred_element_type=jnp.float32)

        @pl.when(cid == 0)
        def _():
            s_ref[0] = sb

        @pl.when(cid == 1)
        def _():
            s_ref[0] = s_ref[0] + sb

    nrb = _NPAD // _R
    return pl.pallas_call(
        body,
        grid=(3, nrb, 2),
        in_specs=[
            pl.BlockSpec((_R, _D), lambda h, r, c: (r, 0)),
            pl.BlockSpec((1, 1, _D, 128), lambda h, r, c: (h, c, 0, 0)),
            pl.BlockSpec((1, 1, 2, 128), lambda h, r, c: (h, c, 0, 0)),
        ],
        out_specs=[
            pl.BlockSpec((1, 1, _R, 128), lambda h, r, c: (h, c, r, 0)),
            pl.BlockSpec((1, 2, _R), lambda h, r, c: (h, 0, r)),
        ],
        out_shape=[
            jax.ShapeDtypeStruct((3, 2, _NPAD, 128), jnp.float32),
            jax.ShapeDtypeStruct((3, 2, _NPAD), jnp.float32),
        ],
    )(x_pad, w_all, a_all)


def _lrelu_exp(z):
    return jnp.exp(jnp.where(z >= 0, z, 0.2 * z))


def _bcast16(j):
    return jnp.zeros((16,), jnp.int32) + j


def _didx(dv):
    return [lax.shift_right_logical(dv, 4), jnp.bitwise_and(dv, 15)]


_MESH = plsc.VectorSubcoreMesh(core_axis_name="c", subcore_axis_name="s",
                               num_cores=_NSC, num_subcores=_NTS)


def _make_alpha_kernel(epad):
    nblk_den = epad // (_NTS * _K)
    nblk_al = epad // (_NSC * _NTS * _K)

    @functools.partial(
        pl.kernel,
        out_type=jax.ShapeDtypeStruct((3 * epad,), jnp.float32),
        mesh=_MESH,
        scratch_types=[
            pltpu.VMEM((_NPAD,), jnp.float32),
            pltpu.VMEM((_NPAD,), jnp.float32),
            pltpu.VMEM((_NPAD,), jnp.float32),
            pltpu.VMEM((_RSLICE,), jnp.float32),
            pltpu.VMEM((_K,), jnp.int32),
            pltpu.VMEM((_K,), jnp.int32),
            pltpu.VMEM((_K,), jnp.float32),
            pltpu.VMEM_SHARED((_NTS * _NPAD,), jnp.float32),
            pltpu.VMEM_SHARED((_NPAD,), jnp.float32),
            pltpu.SemaphoreType.DMA,
        ],
        compiler_params=pltpu.CompilerParams(needs_layout_passes=False),
    )
    def alpha_kernel(sT_hbm, src_hbm, dst_hbm, al_hbm,
                     s_src, s_dst, dloc, tmp, srci, dsti, evb,
                     dparts, dfin, sem):
        cid = lax.axis_index("c")
        sid = lax.axis_index("s")
        rs = sid * _RSLICE

        def per_head(head, carry):
            pltpu.sync_copy(sT_hbm.at[pl.ds(2 * head * _NPAD, _NPAD)], s_src)
            pltpu.sync_copy(sT_hbm.at[pl.ds((2 * head + 1) * _NPAD, _NPAD)],
                            s_dst)

            def zden(i, c):
                dloc[pl.ds(i * 16, 16)] = jnp.zeros((16,), jnp.float32)
                return c
            lax.fori_loop(0, _NPAD // 16, zden, 0)

            def block_a(b, c):
                off = (sid * nblk_den + b) * _K
                pltpu.sync_copy(src_hbm.at[pl.ds(off, _K)], srci)
                pltpu.sync_copy(dst_hbm.at[pl.ds(off, _K)], dsti)
                for i in range(_K // 16):
                    sv = srci[pl.ds(i * 16, 16)]
                    dv = dsti[pl.ds(i * 16, 16)]
                    z = plsc.load_gather(s_src, [sv]) + plsc.load_gather(s_dst, [dv])
                    plsc.addupdate_scatter(dloc, [dv], _lrelu_exp(z))
                return c
            lax.fori_loop(0, nblk_den, block_a, 0)

            pltpu.sync_copy(dloc, dparts.at[pl.ds(sid * _NPAD, _NPAD)])
            plsc.subcore_barrier()

            def zt(i, c):
                tmp[pl.ds(i * 16, 16)] = jnp.zeros((16,), jnp.float32)
                return c
            lax.fori_loop(0, _RSLICE // 16, zt, 0)

            def red(t, c):
                pltpu.sync_copy(dparts.at[pl.ds(t * _NPAD + rs, _RSLICE)],
                                dloc.at[pl.ds(0, _RSLICE)])

                def addt(i, c2):
                    tmp[pl.ds(i * 16, 16)] = (tmp[pl.ds(i * 16, 16)]
                                              + dloc[pl.ds(i * 16, 16)])
                    return c2
                lax.fori_loop(0, _RSLICE // 16, addt, 0)
                return c
            lax.fori_loop(0, _NTS, red, 0)

            def rec(i, c):
                tmp[pl.ds(i * 16, 16)] = 1.0 / tmp[pl.ds(i * 16, 16)]
                return c
            lax.fori_loop(0, _RSLICE // 16, rec, 0)
            pltpu.sync_copy(tmp, dfin.at[pl.ds(rs, _RSLICE)])
            plsc.subcore_barrier()
            pltpu.sync_copy(dfin, dloc)

            def block_b(b, c):
                off = ((cid * _NTS + sid) * nblk_al + b) * _K
                pltpu.sync_copy(src_hbm.at[pl.ds(off, _K)], srci)
                pltpu.sync_copy(dst_hbm.at[pl.ds(off, _K)], dsti)
                for i in range(_K // 16):
                    sv = srci[pl.ds(i * 16, 16)]
                    dv = dsti[pl.ds(i * 16, 16)]
                    z = plsc.load_gather(s_src, [sv]) + plsc.load_gather(s_dst, [dv])
                    evb[pl.ds(i * 16, 16)] = (_lrelu_exp(z)
                                              * plsc.load_gather(dloc, [dv]))
                pltpu.sync_copy(evb, al_hbm.at[pl.ds(head * epad + off, _K)])
                return c
            lax.fori_loop(0, nblk_al, block_b, 0)
            plsc.subcore_barrier()
            return carry

        lax.fori_loop(0, 3, per_head, 0)

    return alpha_kernel


def _make_agg_kernel(epad):
    nblocks = epad // (_NTS * _K)

    @functools.partial(
        pl.kernel,
        out_type=jax.ShapeDtypeStruct((_NPAD, 3 * _H), jnp.float32),
        mesh=_MESH,
        scratch_types=[
            pltpu.VMEM((_K,), jnp.int32),
            pltpu.VMEM((_K,), jnp.int32),
            pltpu.VMEM((_K,), jnp.float32),
            pltpu.VMEM((_K, 128), jnp.float32),
            pltpu.VMEM((_CB, 128), jnp.float32),
            pltpu.VMEM((128,), jnp.float32),
            pltpu.VMEM((128,), jnp.float32),
            pltpu.VMEM_SHARED((_NPAD, 128), jnp.float32),
            pltpu.SemaphoreType.DMA,
        ],
        compiler_params=pltpu.CompilerParams(needs_layout_passes=False),
    )
    def agg_kernel(h_hbm, al_hbm, src_hbm, dst_hbm, p0_hbm, fin_hbm, out_hbm,
                   srci, dsti, ab, rows, cbuf, p0v, finv, acc, sem):
        cid = lax.axis_index("c")
        sid = lax.axis_index("s")
        rs = sid * _RSLICE

        def per_head(head, carry):
            def zcb(j, c):
                for cc in range(8):
                    cbuf[j, pl.ds(cc * 16, 16)] = jnp.zeros((16,), jnp.float32)
                return c
            lax.fori_loop(0, _CB, zcb, 0)

            def zacc(chunk, c):
                pltpu.sync_copy(cbuf, acc.at[pl.ds(rs + chunk * _CB, _CB)])
                return c
            lax.fori_loop(0, _RSLICE // _CB, zacc, 0)
            plsc.subcore_barrier()

            def block_b(b, c):
                off = (sid * nblocks + b) * _K
                pltpu.sync_copy(src_hbm.at[pl.ds(off, _K)], srci)
                pltpu.sync_copy(dst_hbm.at[pl.ds(off, _K)], dsti)
                pltpu.sync_copy(al_hbm.at[pl.ds(head * epad + off, _K)], ab)
                pltpu.async_copy(h_hbm.at[head, cid].at[srci], rows, sem).wait()

                def scale(j, c2):
                    av = plsc.load_gather(ab, [_bcast16(j)])
                    for cc in range(8):
                        rows[j, pl.ds(cc * 16, 16)] = rows[j, pl.ds(cc * 16, 16)] * av
                    return c2
                lax.fori_loop(0, _K, scale, 0)
                pltpu.sync_copy(rows, acc.at[dsti], add=True)
                return c
            lax.fori_loop(0, nblocks, block_b, 0)
            plsc.subcore_barrier()

            pltpu.sync_copy(p0_hbm.at[pl.ds(cid * 128, 128)], p0v)
            pltpu.sync_copy(fin_hbm.at[pl.ds(head * 256 + cid * 128, 128)], finv)
            col0 = head * _H + cid * 128

            def outchunk(chunk, c):
                r0 = rs + chunk * _CB
                pltpu.sync_copy(acc.at[pl.ds(r0, _CB)], cbuf)

                def rowact(j, c2):
                    for cc in range(8):
                        v = cbuf[j, pl.ds(cc * 16, 16)]
                        pa = p0v[pl.ds(cc * 16, 16)]
                        v0 = jnp.where(v >= 0, v, pa * v)
                        v1 = v / (1.0 + jnp.exp(-v))
                        v2 = 1.0 - 2.0 / (1.0 + jnp.exp(2.0 * v))
                        v = jnp.where(head == 0, v0,
                                      jnp.where(head == 1, v1, v2))
                        fa = finv[pl.ds(cc * 16, 16)]
                        v = jnp.where(v >= 0, v, fa * v)
                        cbuf[j, pl.ds(cc * 16, 16)] = v
                    return c2
                lax.fori_loop(0, _CB, rowact, 0)
                pltpu.sync_copy(cbuf,
                                out_hbm.at[pl.ds(r0, _CB), pl.ds(col0, 128)])
                return c
            lax.fori_loop(0, _RSLICE // _CB, outchunk, 0)
            plsc.subcore_barrier()
            return carry

        lax.fori_loop(0, 3, per_head, 0)

    return agg_kernel


def kernel(node_features, edge_index, W0, a_src0, a_dst0, prelu0_alpha,
           W1, a_src1, a_dst1, W2, a_src2, a_dst2, final_prelu_alpha):
    n, d = node_features.shape
    e = edge_index.shape[1]
    etot = e + n
    ealign = _NSC * _NTS * _K
    epad = -(-etot // ealign) * ealign

    loops = jnp.arange(n, dtype=edge_index.dtype)
    src = jnp.concatenate([edge_index[0], loops,
                           jnp.zeros((epad - etot,), edge_index.dtype)])
    dst = jnp.concatenate([edge_index[1], loops,
                           jnp.full((epad - etot,), n, edge_index.dtype)])

    x_pad = jnp.pad(node_features, ((0, _NPAD - n), (0, 0)))
    w_all = jnp.stack([W0, W1, W2]).reshape(3, d, 2, 128).transpose(0, 2, 1, 3)
    a_all = jnp.stack([a_src0, a_dst0, a_src1, a_dst1, a_src2, a_dst2])
    a_all = a_all.reshape(3, 2, 2, 128).transpose(0, 2, 1, 3)

    h_all, sT = _tc_proj(x_pad, w_all, a_all)
    sT = sT.reshape(6 * _NPAD)
    alpha = _make_alpha_kernel(epad)(sT, src, dst)
    out = _make_agg_kernel(epad)(h_all, alpha, src, dst,
                                 prelu0_alpha, final_prelu_alpha)
    return out[:n]
